# baseline (speedup 1.0000x reference)
import functools

import jax
import jax.numpy as jnp
from jax import lax
from jax.experimental import pallas as pl
from jax.experimental.pallas import tpu as pltpu
from jax.experimental.pallas import tpu_sc as plsc

_B, _N, _T = 256, 128, 512
_NTILES = 32
_BPT = _B // _NTILES

_DNUMS = lax.GatherDimensionNumbers(
    offset_dims=(), collapsed_slice_dims=(0,), start_index_map=(0,))


def _shuf(x, idx):
    return lax.gather(x, idx.reshape(16, 1), dimension_numbers=_DNUMS,
                      slice_sizes=(1,),
                      mode=lax.GatherScatterMode.PROMISE_IN_BOUNDS)


def _sc_body(spk_hbm, pot_hbm, out_hbm, pbuf, sbuf, mu_v, nr_v, ab_v,
             cand_v, res_v):
    f32 = jnp.float32
    i32 = jnp.int32
    cid = lax.axis_index("c")
    sid = lax.axis_index("s")
    wid = sid * 2 + cid
    i16 = lax.iota(i32, 16)
    z16 = jnp.zeros((16,), i32)

    def max_splat(x):
        for st in (8, 4, 2, 1):
            x = jnp.maximum(x, _shuf(x, i16 ^ st))
        return x

    def min_splat(x):
        for st in (8, 4, 2, 1):
            x = jnp.minimum(x, _shuf(x, i16 ^ st))
        return x

    def sum_splat(x):
        for st in (8, 4, 2, 1):
            x = x + _shuf(x, i16 ^ st)
        return x

    def prefix_sum(v):
        for st in (1, 2, 4, 8):
            shifted = _shuf(v, jnp.maximum(i16 - st, 0))
            v = v + jnp.where(i16 >= st, shifted, 0)
        return v

    def batch_body(i, _):
        base = (wid * _BPT + i) * _N
        pltpu.sync_copy(pot_hbm.at[pl.ds(base, _N)], pbuf)

        def mu_group(g, _):
            def mu_row(l, accv):
                n = g * 16 + l

                def chunk_max(j, mx):
                    return jnp.maximum(mx, pbuf[n, pl.ds(j * 16, 16)])

                mx = lax.fori_loop(0, _T // 16, chunk_max,
                                   jnp.full((16,), -1e30, f32), unroll=4)
                return jnp.where(i16 == l, max_splat(mx), accv)

            accv = lax.fori_loop(0, 16, mu_row, jnp.zeros((16,), f32))
            mu_v[pl.ds(g * 16, 16)] = accv
            return 0

        lax.fori_loop(0, _N // 16, mu_group, 0)

        def nr_group(g, _):
            u = mu_v[pl.ds(g * 16, 16)]
            gidx = i16 + g * 16

            def nr_j(j, acc):
                w = plsc.load_gather(mu_v, [z16 + j])
                hit = (w > u) | ((w == u) & (j < gidx))
                return acc + jnp.where(hit, 1.0, 0.0)

            acc = lax.fori_loop(0, _N, nr_j, jnp.zeros((16,), f32))
            nr_v[pl.ds(g * 16, 16)] = acc + 1.0
            return 0

        lax.fori_loop(0, _N // 16, nr_group, 0)

        def half_body(h, _):
            pltpu.sync_copy(spk_hbm.at[pl.ds(base + h * 64, 64)], sbuf)

            def row_body(r, _):
                n = h * 64 + r
                nrb = plsc.load_gather(nr_v, [z16 + n])

                def a_chunk(j, mn):
                    sv = sbuf[r, pl.ds(j * 16, 16)]
                    tv = (i16 + (j * 16 + 1)).astype(f32)
                    av = sv * tv + (1.0 - sv) * (512.0 + nrb)
                    ab_v[pl.ds(j * 16, 16)] = av
                    return jnp.minimum(mn, av)

                mnv = lax.fori_loop(0, _T // 16, a_chunk,
                                    jnp.full((16,), 1e30, f32), unroll=4)
                thr = min_splat(mnv) + 5.12

                def cand_chunk(j, off):
                    av = ab_v[pl.ds(j * 16, 16)]
                    msk = av <= thr

                    def compact(o):
                        incl = prefix_sum(jnp.where(msk, 1, 0))
                        pos = o + incl - 1
                        plsc.store_scatter(cand_v, [pos], i16 + j * 16,
                                           mask=msk)
                        return o + _shuf(incl, z16 + 15)

                    return lax.cond(jnp.any(msk), compact, lambda o: o, off)

                nc = lax.fori_loop(0, _T // 16, cand_chunk, z16)

                def ce_cond(st):
                    return jnp.all(st[0] < nc)

                def ce_body(st):
                    kv, best = st
                    ct = plsc.load_gather(cand_v, [kv])
                    pv = plsc.load_gather(pbuf, [z16 + n, ct])
                    sv = plsc.load_gather(sbuf, [z16 + r, ct])

                    def cnt_chunk(j, acc):
                        pc = pbuf[n, pl.ds(j * 16, 16)]
                        return acc + jnp.where(pc > pv, 1.0, 0.0)

                    accv = lax.fori_loop(0, _T // 16, cnt_chunk,
                                         jnp.zeros((16,), f32), unroll=4)
                    trv = sum_splat(accv)
                    ctf = ct.astype(f32)
                    fv = sv * (ctf + 1.0) + (1.0 - sv) * (
                        512.0 + (nrb + 0.01 * trv))
                    return kv + 1, jnp.minimum(best, fv)

                _, best = lax.while_loop(
                    ce_cond, ce_body, (z16, jnp.full((16,), 1e30, f32)))
                plsc.store_scatter(res_v, [z16 + n], best, mask=i16 == 0)
                return 0

            lax.fori_loop(0, 64, row_body, 0)
            return 0

        lax.fori_loop(0, 2, half_body, 0)
        pltpu.sync_copy(res_v, out_hbm.at[pl.ds(base, _N)])
        return 0

    lax.fori_loop(0, _BPT, batch_body, 0)


def kernel(output_spikes, output_potentials):
    B, N, T = output_spikes.shape
    spk = output_spikes.reshape(B * N, T)
    pot = output_potentials.reshape(B * N, T)
    mesh = plsc.VectorSubcoreMesh(core_axis_name="c", subcore_axis_name="s")
    run = functools.partial(
        pl.kernel,
        out_type=jax.ShapeDtypeStruct((B * N,), jnp.float32),
        mesh=mesh,
        compiler_params=pltpu.CompilerParams(needs_layout_passes=False),
        scratch_types=[
            pltpu.VMEM((N, T), jnp.float32),
            pltpu.VMEM((64, T), jnp.float32),
            pltpu.VMEM((N,), jnp.float32),
            pltpu.VMEM((N,), jnp.float32),
            pltpu.VMEM((T,), jnp.float32),
            pltpu.VMEM((T,), jnp.int32),
            pltpu.VMEM((N,), jnp.float32),
        ],
    )(_sc_body)
    out = run(spk, pot)
    return out.reshape(B, N)


# speedup vs baseline: 1.9234x; 1.9234x over previous
import functools

import jax
import jax.numpy as jnp
from jax import lax
from jax.experimental import pallas as pl
from jax.experimental.pallas import tpu as pltpu
from jax.experimental.pallas import tpu_sc as plsc

_B, _N, _T = 256, 128, 512
_NTILES = 32
_BPT = _B // _NTILES
_NCH = _T // 16

_DNUMS = lax.GatherDimensionNumbers(
    offset_dims=(), collapsed_slice_dims=(0,), start_index_map=(0,))


def _shuf(x, idx):
    return lax.gather(x, idx.reshape(16, 1), dimension_numbers=_DNUMS,
                      slice_sizes=(1,),
                      mode=lax.GatherScatterMode.PROMISE_IN_BOUNDS)


def _sc_body(spk_hbm, pot_hbm, out_hbm, pbuf, sbuf, mu_v, nr_v, ab_v,
             tvals_v, cl_lo_v, cl_hi_v, res_v):
    f32 = jnp.float32
    i32 = jnp.int32
    cid = lax.axis_index("c")
    sid = lax.axis_index("s")
    wid = sid * 2 + cid
    i16 = lax.iota(i32, 16)
    z16 = jnp.zeros((16,), i32)

    def max_splat(x):
        for st in (8, 4, 2, 1):
            x = jnp.maximum(x, _shuf(x, i16 ^ st))
        return x

    def min_splat(x):
        for st in (8, 4, 2, 1):
            x = jnp.minimum(x, _shuf(x, i16 ^ st))
        return x

    def sum_splat(x):
        for st in (8, 4, 2, 1):
            x = x + _shuf(x, i16 ^ st)
        return x

    def tv_init(j, _):
        tvals_v[pl.ds(j * 16, 16)] = (i16 + (j * 16 + 1)).astype(f32)
        return 0

    lax.fori_loop(0, _NCH, tv_init, 0)

    def batch_body(i, _):
        base = (wid * _BPT + i) * _N
        pltpu.sync_copy(pot_hbm.at[pl.ds(base, _N)], pbuf)

        def mu_group(g, _):
            def mu_row(l, accv):
                n = g * 16 + l

                def chunk_max(j, mx):
                    return jnp.maximum(mx, pbuf[n, pl.ds(j * 16, 16)])

                mx = lax.fori_loop(0, _NCH, chunk_max,
                                   jnp.full((16,), -1e30, f32))
                return jnp.where(i16 == l, max_splat(mx), accv)

            accv = lax.fori_loop(0, 16, mu_row, jnp.zeros((16,), f32))
            mu_v[pl.ds(g * 16, 16)] = accv
            return 0

        lax.fori_loop(0, _N // 16, mu_group, 0)

        def nr_group(g, _):
            u = mu_v[pl.ds(g * 16, 16)]
            gidx = i16 + g * 16

            def nr_j(j, acc):
                w = plsc.load_gather(mu_v, [z16 + j])
                hit = (w > u) | ((w == u) & (j < gidx))
                return acc + jnp.where(hit, 1.0, 0.0)

            acc = lax.fori_loop(0, _N, nr_j, jnp.zeros((16,), f32))
            nr_v[pl.ds(g * 16, 16)] = acc + 1.0
            return 0

        lax.fori_loop(0, _N // 16, nr_group, 0)

        def half_body(h, _):
            pltpu.sync_copy(spk_hbm.at[pl.ds(base + h * 64, 64)], sbuf)

            def row_body(r, _):
                n = h * 64 + r
                nrb = plsc.load_gather(nr_v, [z16 + n])
                c512 = 512.0 + nrb

                def a_chunk(j, mn):
                    sv = sbuf[r, pl.ds(j * 16, 16)]
                    tv = tvals_v[pl.ds(j * 16, 16)]
                    av = sv * tv + (1.0 - sv) * c512
                    ab_v[pl.ds(j * 16, 16)] = av
                    return jnp.minimum(mn, av)

                mnv = lax.fori_loop(0, _NCH, a_chunk,
                                    jnp.full((16,), 1e30, f32))
                thr = min_splat(mnv) + 5.12

                cm_lo = jnp.full((16,), 1e30, f32)
                cm_hi = jnp.full((16,), 1e30, f32)
                for l in range(16):
                    cm_lo = jnp.minimum(
                        cm_lo, plsc.load_gather(ab_v, [i16 * 16 + l]))
                    cm_hi = jnp.minimum(
                        cm_hi, plsc.load_gather(ab_v, [i16 * 16 + (256 + l)]))
                msk_lo = cm_lo <= thr
                msk_hi = cm_hi <= thr
                plsc.store_compressed(cl_lo_v.at[pl.ds(0, 16)], i16, mask=msk_lo)
                plsc.store_compressed(cl_hi_v.at[pl.ds(0, 16)], i16 + 16, mask=msk_hi)
                ncl = plsc.all_reduce_population_count(msk_lo)
                nch = plsc.all_reduce_population_count(msk_hi)

                def eval_cand(ct, best):
                    pv = plsc.load_gather(pbuf, [z16 + n, ct])
                    sv = plsc.load_gather(sbuf, [z16 + r, ct])

                    def cnt_chunk(j, acc):
                        pc = pbuf[n, pl.ds(j * 16, 16)]
                        return acc + jnp.where(pc > pv, 1.0, 0.0)

                    accv = lax.fori_loop(0, _NCH, cnt_chunk,
                                         jnp.zeros((16,), f32))
                    trv = sum_splat(accv)
                    ctf = ct.astype(f32)
                    fv = sv * (ctf + 1.0) + (1.0 - sv) * (
                        512.0 + (nrb + 0.01 * trv))
                    return jnp.minimum(best, fv)

                best = jnp.full((16,), 1e30, f32)
                for clref, ncnt in ((cl_lo_v, ncl), (cl_hi_v, nch)):
                    def o_cond(st):
                        return jnp.all(st[0] < ncnt)

                    def o_body(st):
                        cv, best = st
                        jv = plsc.load_gather(clref, [cv])
                        avj = plsc.load_gather(ab_v, [jv * 16 + i16])
                        mskj = avj <= thr

                        def i_cond(st2):
                            return jnp.any(st2[0])

                        def i_body(st2):
                            msk2, best2 = st2
                            lv = plsc.all_reduce_ffs(msk2)
                            ct = jv * 16 + lv
                            best2 = eval_cand(ct, best2)
                            return msk2 & (i16 != lv), best2

                        _, best = lax.while_loop(i_cond, i_body, (mskj, best))
                        return cv + 1, best

                    _, best = lax.while_loop(o_cond, o_body, (z16, best))
                plsc.store_scatter(res_v, [z16 + n], best, mask=i16 == 0)
                return 0

            lax.fori_loop(0, 64, row_body, 0)
            return 0

        lax.fori_loop(0, 2, half_body, 0)
        pltpu.sync_copy(res_v, out_hbm.at[pl.ds(base, _N)])
        return 0

    lax.fori_loop(0, _BPT, batch_body, 0)


def kernel(output_spikes, output_potentials):
    B, N, T = output_spikes.shape
    spk = output_spikes.reshape(B * N, T)
    pot = output_potentials.reshape(B * N, T)
    mesh = plsc.VectorSubcoreMesh(core_axis_name="c", subcore_axis_name="s")
    run = functools.partial(
        pl.kernel,
        out_type=jax.ShapeDtypeStruct((B * N,), jnp.float32),
        mesh=mesh,
        compiler_params=pltpu.CompilerParams(needs_layout_passes=False),
        scratch_types=[
            pltpu.VMEM((N, T), jnp.float32),
            pltpu.VMEM((64, T), jnp.float32),
            pltpu.VMEM((N,), jnp.float32),
            pltpu.VMEM((N,), jnp.float32),
            pltpu.VMEM((T,), jnp.float32),
            pltpu.VMEM((T,), jnp.float32),
            pltpu.VMEM((16,), jnp.int32),
            pltpu.VMEM((16,), jnp.int32),
            pltpu.VMEM((N,), jnp.float32),
        ],
    )(_sc_body)
    out = run(spk, pot)
    return out.reshape(B, N)


# speedup vs baseline: 2.1528x; 1.1193x over previous
import functools

import jax
import jax.numpy as jnp
from jax import lax
from jax.experimental import pallas as pl
from jax.experimental.pallas import tpu as pltpu
from jax.experimental.pallas import tpu_sc as plsc

_B, _N, _T = 256, 128, 512
_NTILES = 32
_BPT = _B // _NTILES
_NCH = _T // 16

_DNUMS = lax.GatherDimensionNumbers(
    offset_dims=(), collapsed_slice_dims=(0,), start_index_map=(0,))


def _shuf(x, idx):
    return lax.gather(x, idx.reshape(16, 1), dimension_numbers=_DNUMS,
                      slice_sizes=(1,),
                      mode=lax.GatherScatterMode.PROMISE_IN_BOUNDS)


def _sc_body(spk_hbm, pot_hbm, out_hbm, pbuf, sbuf, mu_v, nr_v, ab_v,
             tvals_v, cl_lo_v, cl_hi_v, res_v):
    f32 = jnp.float32
    i32 = jnp.int32
    cid = lax.axis_index("c")
    sid = lax.axis_index("s")
    wid = sid * 2 + cid
    i16 = lax.iota(i32, 16)
    z16 = jnp.zeros((16,), i32)

    def max_splat(x):
        for st in (8, 4, 2, 1):
            x = jnp.maximum(x, _shuf(x, i16 ^ st))
        return x

    def min_splat(x):
        for st in (8, 4, 2, 1):
            x = jnp.minimum(x, _shuf(x, i16 ^ st))
        return x

    def sum_splat(x):
        for st in (8, 4, 2, 1):
            x = x + _shuf(x, i16 ^ st)
        return x

    def tv_init(j, _):
        tvals_v[pl.ds(j * 16, 16)] = (i16 + (j * 16 + 1)).astype(f32)
        return 0

    lax.fori_loop(0, _NCH, tv_init, 0)

    def batch_body(i, _):
        base = (wid * _BPT + i) * _N
        pltpu.sync_copy(pot_hbm.at[pl.ds(base, _N)], pbuf)

        def mu_group(g, _):
            def mu_row(l, accv):
                n = g * 16 + l

                def chunk_max(j, mx):
                    return jnp.maximum(mx, pbuf[n, pl.ds(j * 16, 16)])

                mx = lax.fori_loop(0, _NCH, chunk_max,
                                   jnp.full((16,), -1e30, f32))
                return jnp.where(i16 == l, max_splat(mx), accv)

            accv = lax.fori_loop(0, 16, mu_row, jnp.zeros((16,), f32))
            mu_v[pl.ds(g * 16, 16)] = accv
            return 0

        lax.fori_loop(0, _N // 16, mu_group, 0)

        def nr_group(g, _):
            u = mu_v[pl.ds(g * 16, 16)]
            gidx = i16 + g * 16

            def nr_j(j, acc):
                w = plsc.load_gather(mu_v, [z16 + j])
                hit = (w > u) | ((w == u) & (j < gidx))
                return acc + jnp.where(hit, 1.0, 0.0)

            acc = lax.fori_loop(0, _N, nr_j, jnp.zeros((16,), f32))
            nr_v[pl.ds(g * 16, 16)] = acc + 1.0
            return 0

        lax.fori_loop(0, _N // 16, nr_group, 0)

        def half_body(h, _):
            pltpu.sync_copy(spk_hbm.at[pl.ds(base + h * 64, 64)], sbuf)

            def row_body(r, _):
                n = h * 64 + r
                nrb = plsc.load_gather(nr_v, [z16 + n])
                c512 = 512.0 + nrb

                def a_chunk(j, mn):
                    sv = sbuf[r, pl.ds(j * 32, 16)]
                    tv = tvals_v[pl.ds(j * 32, 16)]
                    av = sv * tv + (1.0 - sv) * c512
                    ab_v[pl.ds(j * 32, 16)] = av
                    sv2 = sbuf[r, pl.ds(j * 32 + 16, 16)]
                    tv2 = tvals_v[pl.ds(j * 32 + 16, 16)]
                    av2 = sv2 * tv2 + (1.0 - sv2) * c512
                    ab_v[pl.ds(j * 32 + 16, 16)] = av2
                    return jnp.minimum(mn, jnp.minimum(av, av2))

                mnv = lax.fori_loop(0, _NCH // 2, a_chunk,
                                    jnp.full((16,), 1e30, f32))
                thr = min_splat(mnv) + 5.12

                cm_lo = jnp.full((16,), 1e30, f32)
                cm_hi = jnp.full((16,), 1e30, f32)
                for l in range(16):
                    cm_lo = jnp.minimum(
                        cm_lo, plsc.load_gather(ab_v, [i16 * 16 + l]))
                    cm_hi = jnp.minimum(
                        cm_hi, plsc.load_gather(ab_v, [i16 * 16 + (256 + l)]))
                msk_lo = cm_lo <= thr
                msk_hi = cm_hi <= thr
                plsc.store_compressed(cl_lo_v.at[pl.ds(0, 16)], i16, mask=msk_lo)
                plsc.store_compressed(cl_hi_v.at[pl.ds(0, 16)], i16 + 16, mask=msk_hi)
                ncl = plsc.all_reduce_population_count(msk_lo)
                nch = plsc.all_reduce_population_count(msk_hi)

                def eval_cand(ct, best):
                    pv = plsc.load_gather(pbuf, [z16 + n, ct])
                    sv = plsc.load_gather(sbuf, [z16 + r, ct])

                    def cnt_chunk(j, acc):
                        pc = pbuf[n, pl.ds(j * 32, 16)]
                        pc2 = pbuf[n, pl.ds(j * 32 + 16, 16)]
                        return (acc + jnp.where(pc > pv, 1.0, 0.0)
                                + jnp.where(pc2 > pv, 1.0, 0.0))

                    accv = lax.fori_loop(0, _NCH // 2, cnt_chunk,
                                         jnp.zeros((16,), f32))
                    trv = sum_splat(accv)
                    ctf = ct.astype(f32)
                    fv = sv * (ctf + 1.0) + (1.0 - sv) * (
                        512.0 + (nrb + 0.01 * trv))
                    return jnp.minimum(best, fv)

                best = jnp.full((16,), 1e30, f32)
                for clref, ncnt in ((cl_lo_v, ncl), (cl_hi_v, nch)):
                    def o_cond(st):
                        return jnp.all(st[0] < ncnt)

                    def o_body(st):
                        cv, best = st
                        jv = plsc.load_gather(clref, [cv])
                        avj = plsc.load_gather(ab_v, [jv * 16 + i16])
                        mskj = avj <= thr

                        def i_cond(st2):
                            return jnp.any(st2[0])

                        def i_body(st2):
                            msk2, best2 = st2
                            lv = plsc.all_reduce_ffs(msk2)
                            ct = jv * 16 + lv
                            best2 = eval_cand(ct, best2)
                            return msk2 & (i16 != lv), best2

                        _, best = lax.while_loop(i_cond, i_body, (mskj, best))
                        return cv + 1, best

                    _, best = lax.while_loop(o_cond, o_body, (z16, best))
                plsc.store_scatter(res_v, [z16 + n], best, mask=i16 == 0)
                return 0

            lax.fori_loop(0, 64, row_body, 0)
            return 0

        lax.fori_loop(0, 2, half_body, 0)
        pltpu.sync_copy(res_v, out_hbm.at[pl.ds(base, _N)])
        return 0

    lax.fori_loop(0, _BPT, batch_body, 0)


def kernel(output_spikes, output_potentials):
    B, N, T = output_spikes.shape
    spk = output_spikes.reshape(B * N, T)
    pot = output_potentials.reshape(B * N, T)
    mesh = plsc.VectorSubcoreMesh(core_axis_name="c", subcore_axis_name="s")
    run = functools.partial(
        pl.kernel,
        out_type=jax.ShapeDtypeStruct((B * N,), jnp.float32),
        mesh=mesh,
        compiler_params=pltpu.CompilerParams(needs_layout_passes=False),
        scratch_types=[
            pltpu.VMEM((N, T), jnp.float32),
            pltpu.VMEM((64, T), jnp.float32),
            pltpu.VMEM((N,), jnp.float32),
            pltpu.VMEM((N,), jnp.float32),
            pltpu.VMEM((T,), jnp.float32),
            pltpu.VMEM((T,), jnp.float32),
            pltpu.VMEM((16,), jnp.int32),
            pltpu.VMEM((16,), jnp.int32),
            pltpu.VMEM((N,), jnp.float32),
        ],
    )(_sc_body)
    out = run(spk, pot)
    return out.reshape(B, N)


# speedup vs baseline: 2.4019x; 1.1157x over previous
import functools

import jax
import jax.numpy as jnp
from jax import lax
from jax.experimental import pallas as pl
from jax.experimental.pallas import tpu as pltpu
from jax.experimental.pallas import tpu_sc as plsc

_B, _N, _T = 256, 128, 512
_NTILES = 32
_BPT = _B // _NTILES
_NCH = _T // 16

_DNUMS = lax.GatherDimensionNumbers(
    offset_dims=(), collapsed_slice_dims=(0,), start_index_map=(0,))


def _shuf(x, idx):
    return lax.gather(x, idx.reshape(16, 1), dimension_numbers=_DNUMS,
                      slice_sizes=(1,),
                      mode=lax.GatherScatterMode.PROMISE_IN_BOUNDS)


def _sc_body(spk_hbm, pot_hbm, out_hbm, pbuf, sbuf, mu_v, nr_v, ab_v,
             tvals_v, cl_lo_v, cl_hi_v, res_v):
    f32 = jnp.float32
    i32 = jnp.int32
    cid = lax.axis_index("c")
    sid = lax.axis_index("s")
    wid = sid * 2 + cid
    i16 = lax.iota(i32, 16)
    z16 = jnp.zeros((16,), i32)

    def max_splat(x):
        for st in (8, 4, 2, 1):
            x = jnp.maximum(x, _shuf(x, i16 ^ st))
        return x

    def min_splat(x):
        for st in (8, 4, 2, 1):
            x = jnp.minimum(x, _shuf(x, i16 ^ st))
        return x

    def sum_splat(x):
        for st in (8, 4, 2, 1):
            x = x + _shuf(x, i16 ^ st)
        return x

    def tv_init(j, _):
        tvals_v[pl.ds(j * 16, 16)] = (i16 + (j * 16 + 1)).astype(f32)
        return 0

    lax.fori_loop(0, _NCH, tv_init, 0)

    def batch_body(i, _):
        base = (wid * _BPT + i) * _N
        pltpu.sync_copy(pot_hbm.at[pl.ds(base, _N)], pbuf)

        def mu_group(g, _):
            def mu_row(l, accv):
                n = g * 16 + l

                def chunk_max(j, mx):
                    m1 = jnp.maximum(mx, pbuf[n, pl.ds(j * 32, 16)])
                    return jnp.maximum(m1, pbuf[n, pl.ds(j * 32 + 16, 16)])

                mx = lax.fori_loop(0, _NCH // 2, chunk_max,
                                   jnp.full((16,), -1e30, f32))
                return jnp.where(i16 == l, max_splat(mx), accv)

            accv = lax.fori_loop(0, 16, mu_row, jnp.zeros((16,), f32))
            mu_v[pl.ds(g * 16, 16)] = accv
            return 0

        lax.fori_loop(0, _N // 16, mu_group, 0)

        def nr_group(g, _):
            u = mu_v[pl.ds(g * 16, 16)]
            gidx = i16 + g * 16

            def nr_j(j, acc):
                w = plsc.load_gather(mu_v, [z16 + j])
                hit = (w > u) | ((w == u) & (j < gidx))
                return acc + jnp.where(hit, 1.0, 0.0)

            acc = lax.fori_loop(0, _N, nr_j, jnp.zeros((16,), f32))
            nr_v[pl.ds(g * 16, 16)] = acc + 1.0
            return 0

        lax.fori_loop(0, _N // 16, nr_group, 0)

        def half_body(h, _):
            pltpu.sync_copy(spk_hbm.at[pl.ds(base + h * 64, 64)], sbuf)

            def row_body(r, _):
                n = h * 64 + r
                nrb = plsc.load_gather(nr_v, [z16 + n])
                c512 = 512.0 + nrb

                def a_chunk(j, mn):
                    sv = sbuf[r, pl.ds(j * 32, 16)]
                    tv = tvals_v[pl.ds(j * 32, 16)]
                    av = sv * tv + (1.0 - sv) * c512
                    ab_v[pl.ds(j * 32, 16)] = av
                    sv2 = sbuf[r, pl.ds(j * 32 + 16, 16)]
                    tv2 = tvals_v[pl.ds(j * 32 + 16, 16)]
                    av2 = sv2 * tv2 + (1.0 - sv2) * c512
                    ab_v[pl.ds(j * 32 + 16, 16)] = av2
                    return jnp.minimum(mn, jnp.minimum(av, av2))

                mnv = lax.fori_loop(0, _NCH // 2, a_chunk,
                                    jnp.full((16,), 1e30, f32))
                thr = min_splat(mnv) + 5.12

                cm_lo = jnp.full((16,), 1e30, f32)
                cm_hi = jnp.full((16,), 1e30, f32)
                for l in range(16):
                    cm_lo = jnp.minimum(
                        cm_lo, plsc.load_gather(ab_v, [i16 * 16 + l]))
                    cm_hi = jnp.minimum(
                        cm_hi, plsc.load_gather(ab_v, [i16 * 16 + (256 + l)]))
                msk_lo = cm_lo <= thr
                msk_hi = cm_hi <= thr
                plsc.store_compressed(cl_lo_v.at[pl.ds(0, 16)], i16, mask=msk_lo)
                plsc.store_compressed(cl_hi_v.at[pl.ds(0, 16)], i16 + 16, mask=msk_hi)
                ncl = plsc.all_reduce_population_count(msk_lo)
                nch = plsc.all_reduce_population_count(msk_hi)

                def eval_cand(ct, best):
                    pv = plsc.load_gather(pbuf, [z16 + n, ct])
                    sv = plsc.load_gather(sbuf, [z16 + r, ct])

                    def cnt_chunk(j, acc):
                        pc = pbuf[n, pl.ds(j * 32, 16)]
                        pc2 = pbuf[n, pl.ds(j * 32 + 16, 16)]
                        return (acc + jnp.where(pc > pv, 1.0, 0.0)
                                + jnp.where(pc2 > pv, 1.0, 0.0))

                    accv = lax.fori_loop(0, _NCH // 2, cnt_chunk,
                                         jnp.zeros((16,), f32))
                    trv = sum_splat(accv)
                    ctf = ct.astype(f32)
                    fv = sv * (ctf + 1.0) + (1.0 - sv) * (
                        512.0 + (nrb + 0.01 * trv))
                    return jnp.minimum(best, fv)

                best = jnp.full((16,), 1e30, f32)
                for clref, ncnt in ((cl_lo_v, ncl), (cl_hi_v, nch)):
                    def o_cond(st):
                        return jnp.all(st[0] < ncnt)

                    def o_body(st):
                        cv, best = st
                        jv = plsc.load_gather(clref, [cv])
                        avj = plsc.load_gather(ab_v, [jv * 16 + i16])
                        mskj = avj <= thr

                        def i_cond(st2):
                            return jnp.any(st2[0])

                        def i_body(st2):
                            msk2, best2 = st2
                            lv = plsc.all_reduce_ffs(msk2)
                            ct = jv * 16 + lv
                            best2 = eval_cand(ct, best2)
                            return msk2 & (i16 != lv), best2

                        _, best = lax.while_loop(i_cond, i_body, (mskj, best))
                        return cv + 1, best

                    _, best = lax.while_loop(o_cond, o_body, (z16, best))
                plsc.store_scatter(res_v, [z16 + n], best, mask=i16 == 0)
                return 0

            lax.fori_loop(0, 64, row_body, 0)
            return 0

        lax.fori_loop(0, 2, half_body, 0)
        pltpu.sync_copy(res_v, out_hbm.at[pl.ds(base, _N)])
        return 0

    lax.fori_loop(0, _BPT, batch_body, 0)


def kernel(output_spikes, output_potentials):
    B, N, T = output_spikes.shape
    spk = output_spikes.reshape(B * N, T)
    pot = output_potentials.reshape(B * N, T)
    mesh = plsc.VectorSubcoreMesh(core_axis_name="c", subcore_axis_name="s")
    run = functools.partial(
        pl.kernel,
        out_type=jax.ShapeDtypeStruct((B * N,), jnp.float32),
        mesh=mesh,
        compiler_params=pltpu.CompilerParams(needs_layout_passes=False),
        scratch_types=[
            pltpu.VMEM((N, T), jnp.float32),
            pltpu.VMEM((64, T), jnp.float32),
            pltpu.VMEM((N,), jnp.float32),
            pltpu.VMEM((N,), jnp.float32),
            pltpu.VMEM((T,), jnp.float32),
            pltpu.VMEM((T,), jnp.float32),
            pltpu.VMEM((16,), jnp.int32),
            pltpu.VMEM((16,), jnp.int32),
            pltpu.VMEM((N,), jnp.float32),
        ],
    )(_sc_body)
    out = run(spk, pot)
    return out.reshape(B, N)
